# Initial kernel scaffold; baseline (speedup 1.0000x reference)
import jax
import jax.numpy as jnp
from jax.experimental import pallas as pl


def kernel(x, edge_index, edge_attr, rel_edge_index, rel_edge_type, params):
    raise NotImplementedError("write your pallas kernel here")


# speedup vs baseline: 3.8816x; 3.8816x over previous
import functools

import jax
import jax.numpy as jnp
from jax import lax
from jax.experimental import pallas as pl
from jax.experimental.pallas import tpu as pltpu
from jax.experimental.pallas import tpu_sc as plsc

N = 10000
E = 320000
DF = 128
ND = 64
ED = 16
R = 4

NC = 2
NS = 16
NW = NC * NS
LROW = 128
ROWS = E // LROW

PER_S_N = N // NS
PER_S_2N = 2 * N // NS
PER_S_4N = 4 * N // NS
ZB = 125

_mesh = lambda: plsc.VectorSubcoreMesh(core_axis_name="c", subcore_axis_name="s")
_SC_PARAMS = pltpu.CompilerParams(use_tc_tiling_on_sc=False)


def _worker_rows(w, nworkers):
    start = (ROWS * w) // nworkers
    end = (ROWS * (w + 1)) // nworkers
    return start, end


def _zero_table(zb, table, s, rows_per_tile, zcols):
    def zb_body(i, carry):
        for off in range(0, zcols, 16):
            zb[i, pl.ds(off, 16)] = jnp.zeros((16,), jnp.float32)
        return carry
    lax.fori_loop(0, ZB, zb_body, 0)
    nchunks = rows_per_tile // ZB
    base = s * rows_per_tile
    def cp_body(k, carry):
        pltpu.sync_copy(zb.at[pl.ds(0, ZB)], table.at[pl.ds(base + k * ZB, ZB)])
        return carry
    lax.fori_loop(0, nchunks, cp_body, 0)


def _make_sc_counts():
    @functools.partial(
        pl.kernel,
        out_type=(jax.ShapeDtypeStruct((NC * N, ED), jnp.float32),
                  jax.ShapeDtypeStruct((NC * 4 * N, ED), jnp.float32)),
        mesh=_mesh(),
        compiler_params=_SC_PARAMS,
        scratch_types=[
            pltpu.VMEM((1, LROW), jnp.int32),
            pltpu.VMEM((1, LROW), jnp.int32),
            pltpu.VMEM((1, LROW), jnp.int32),
            pltpu.VMEM((1, LROW), jnp.int32),
            pltpu.VMEM((LROW, ED), jnp.float32),
            pltpu.VMEM((ZB, ED), jnp.float32),
            pltpu.VMEM_SHARED((N, ED), jnp.float32),
            pltpu.VMEM_SHARED((4 * N, ED), jnp.float32),
        ],
    )
    def sc_counts(dst_h, rdst_h, rtyp_h, ocnt_h, ocrel_h,
                  db, rdb, rtb, sidb, ones, zb, cntT, crelT):
        c = lax.axis_index("c")
        s = lax.axis_index("s")
        w = s * NC + c
        _zero_table(zb, cntT, s, PER_S_N, ED)
        _zero_table(zb, crelT, s, PER_S_4N, ED)
        pat = jnp.where(jnp.arange(16, dtype=jnp.int32) == 0, 1.0, 0.0)
        def ones_body(i, carry):
            ones[i, :] = pat
            return carry
        lax.fori_loop(0, LROW, ones_body, 0)
        plsc.subcore_barrier()

        start, end = _worker_rows(w, NW)
        def body(r, carry):
            pltpu.sync_copy(dst_h.at[pl.ds(r, 1)], db)
            pltpu.sync_copy(rdst_h.at[pl.ds(r, 1)], rdb)
            pltpu.sync_copy(rtyp_h.at[pl.ds(r, 1)], rtb)
            for j in range(0, LROW, 16):
                t = rtb[0, pl.ds(j, 16)]
                d = rdb[0, pl.ds(j, 16)]
                sidb[0, pl.ds(j, 16)] = t * N + d
            pltpu.sync_copy(ones, cntT.at[db.at[0]], add=True)
            pltpu.sync_copy(ones, crelT.at[sidb.at[0]], add=True)
            return carry
        lax.fori_loop(start, end, body, 0)
        plsc.subcore_barrier()

        pltpu.sync_copy(cntT.at[pl.ds(s * PER_S_N, PER_S_N)],
                        ocnt_h.at[pl.ds(c * N + s * PER_S_N, PER_S_N)])
        pltpu.sync_copy(crelT.at[pl.ds(s * PER_S_4N, PER_S_4N)],
                        ocrel_h.at[pl.ds(c * 4 * N + s * PER_S_4N, PER_S_4N)])
    return sc_counts


def _make_sc_msum():
    @functools.partial(
        pl.kernel,
        out_type=jax.ShapeDtypeStruct((NC * N, ND), jnp.float32),
        mesh=_mesh(),
        compiler_params=_SC_PARAMS,
        scratch_types=[
            pltpu.VMEM((1, LROW), jnp.int32),
            pltpu.VMEM((1, LROW), jnp.int32),
            pltpu.VMEM((LROW, ND), jnp.float32),
            pltpu.VMEM((LROW, ND), jnp.float32),
            pltpu.VMEM((ZB, ND), jnp.float32),
            pltpu.VMEM_SHARED((N, ND), jnp.float32),
        ],
    )
    def sc_msum(xm_h, em_h, src_h, dst_h, out_h, sb, db, xr, er, zb, msum):
        c = lax.axis_index("c")
        s = lax.axis_index("s")
        w = s * NC + c
        _zero_table(zb, msum, s, PER_S_N, ND)
        plsc.subcore_barrier()

        start, end = _worker_rows(w, NW)
        def body(r, carry):
            pltpu.sync_copy(src_h.at[pl.ds(r, 1)], sb)
            pltpu.sync_copy(dst_h.at[pl.ds(r, 1)], db)
            pltpu.sync_copy(xm_h.at[sb.at[0]], xr)
            pltpu.sync_copy(em_h.at[pl.ds(r * LROW, LROW)], er)
            def cbody(i, carry2):
                for off in range(0, ND, 16):
                    a = xr[i, pl.ds(off, 16)]
                    b = er[i, pl.ds(off, 16)]
                    xr[i, pl.ds(off, 16)] = jnp.maximum(a + b, 0.0)
                return carry2
            lax.fori_loop(0, LROW, cbody, 0)
            pltpu.sync_copy(xr, msum.at[db.at[0]], add=True)
            return carry
        lax.fori_loop(start, end, body, 0)
        plsc.subcore_barrier()

        pltpu.sync_copy(msum.at[pl.ds(s * PER_S_N, PER_S_N)],
                        out_h.at[pl.ds(c * N + s * PER_S_N, PER_S_N)])
    return sc_msum


def _make_sc_rel():
    ZROWS = 2 * N + NS
    @functools.partial(
        pl.kernel,
        out_type=jax.ShapeDtypeStruct((NC * 2 * N, ND), jnp.float32),
        mesh=_mesh(),
        compiler_params=_SC_PARAMS,
        scratch_types=[
            pltpu.VMEM((1, LROW), jnp.int32),
            pltpu.VMEM((1, LROW), jnp.int32),
            pltpu.VMEM((1, LROW), jnp.int32),
            pltpu.VMEM((1, LROW), jnp.int32),
            pltpu.VMEM((LROW, ND), jnp.float32),
            pltpu.VMEM((ZB, ND), jnp.float32),
            pltpu.VMEM_SHARED((ZROWS, ND), jnp.float32),
        ],
    )
    def sc_rel(x_h, rsrc_h, rdst_h, rtyp_h, out_h, sb, db, tb, sidb, xr, zb, Zt):
        c = lax.axis_index("c")
        s = lax.axis_index("s")
        _zero_table(zb, Zt, s, PER_S_2N, ND)
        @pl.when(s == 0)
        def _():
            pltpu.sync_copy(zb.at[pl.ds(0, NS)], Zt.at[pl.ds(2 * N, NS)])
        plsc.subcore_barrier()

        trash = 2 * N + s
        start = (ROWS * s) // NS
        end = (ROWS * (s + 1)) // NS
        def body(r, carry):
            pltpu.sync_copy(rsrc_h.at[pl.ds(r, 1)], sb)
            pltpu.sync_copy(rdst_h.at[pl.ds(r, 1)], db)
            pltpu.sync_copy(rtyp_h.at[pl.ds(r, 1)], tb)
            for j in range(0, LROW, 16):
                t = tb[0, pl.ds(j, 16)]
                d = db[0, pl.ds(j, 16)]
                keep = (t >> 1) == c
                sidb[0, pl.ds(j, 16)] = jnp.where(keep, (t & 1) * N + d, trash)
            pltpu.sync_copy(x_h.at[sb.at[0]], xr)
            pltpu.sync_copy(xr, Zt.at[sidb.at[0]], add=True)
            return carry
        lax.fori_loop(start, end, body, 0)
        plsc.subcore_barrier()

        pltpu.sync_copy(Zt.at[pl.ds(s * PER_S_2N, PER_S_2N)],
                        out_h.at[pl.ds(c * 2 * N + s * PER_S_2N, PER_S_2N)])
    return sc_rel


def _make_sc_ea():
    @functools.partial(
        pl.kernel,
        out_type=jax.ShapeDtypeStruct((E, ED), jnp.float32),
        mesh=_mesh(),
        compiler_params=_SC_PARAMS,
        scratch_types=[
            pltpu.VMEM((1, LROW), jnp.int32),
            pltpu.VMEM((1, LROW), jnp.int32),
            pltpu.VMEM((LROW, ED), jnp.float32),
            pltpu.VMEM((LROW, ED), jnp.float32),
            pltpu.VMEM((LROW, ED), jnp.float32),
        ],
    )
    def sc_ea(u_h, v_h, te_h, src_h, dst_h, out_h, sb, db, ur, vr, tr):
        c = lax.axis_index("c")
        s = lax.axis_index("s")
        w = s * NC + c
        start, end = _worker_rows(w, NW)
        def body(r, carry):
            pltpu.sync_copy(src_h.at[pl.ds(r, 1)], sb)
            pltpu.sync_copy(dst_h.at[pl.ds(r, 1)], db)
            pltpu.sync_copy(u_h.at[sb.at[0]], ur)
            pltpu.sync_copy(v_h.at[db.at[0]], vr)
            pltpu.sync_copy(te_h.at[pl.ds(r * LROW, LROW)], tr)
            def cbody(i, carry2):
                ur[i, :] = jnp.maximum(ur[i, :] + vr[i, :] + tr[i, :], 0.0)
                return carry2
            lax.fori_loop(0, LROW, cbody, 0)
            pltpu.sync_copy(ur, out_h.at[pl.ds(r * LROW, LROW)])
            return carry
        lax.fori_loop(start, end, body, 0)
    return sc_ea


_sc_counts = _make_sc_counts()
_sc_msum = _make_sc_msum()
_sc_rel = _make_sc_rel()
_sc_ea = _make_sc_ea()


def _mm_body(x_ref, w_ref, o_ref):
    o_ref[...] = jnp.dot(x_ref[...], w_ref[...],
                         preferred_element_type=jnp.float32)


def _tc_mm(x, w):
    m, k = x.shape
    _, n = w.shape
    return pl.pallas_call(
        _mm_body,
        out_shape=jax.ShapeDtypeStruct((m, n), jnp.float32),
    )(x, w)


def _edge_mm_body(a_ref, w_ref, b_ref, o_ref):
    o_ref[...] = jnp.dot(a_ref[...], w_ref[...],
                         preferred_element_type=jnp.float32) + b_ref[...]


def _tc_edge_mm(a, w, b):
    BE = 8000
    n = w.shape[1]
    return pl.pallas_call(
        _edge_mm_body,
        grid=(E // BE,),
        in_specs=[
            pl.BlockSpec((BE, ED), lambda i: (i, 0)),
            pl.BlockSpec((ED, n), lambda i: (0, 0)),
            pl.BlockSpec((1, n), lambda i: (0, 0)),
        ],
        out_specs=pl.BlockSpec((BE, n), lambda i: (i, 0)),
        out_shape=jax.ShapeDtypeStruct((E, n), jnp.float32),
    )(a, w, b)


def _aggup_body(ms_ref, cnt_ref, x_ref, wa1_ref, wa2_ref, ba_ref, o_ref):
    c2 = cnt_ref[...]
    cnt = c2[0, :, 0:1] + c2[1, :, 0:1]
    rin = 1.0 / jnp.maximum(cnt, 1.0)
    ms = ms_ref[...]
    agg = (ms[0] + ms[1]) * rin
    o_ref[...] = jnp.maximum(
        jnp.dot(agg, wa1_ref[...], preferred_element_type=jnp.float32)
        + jnp.dot(x_ref[...], wa2_ref[...], preferred_element_type=jnp.float32)
        + ba_ref[...], 0.0)


def _tc_aggup(ms, cnt2, x, wa1, wa2, ba):
    return pl.pallas_call(
        _aggup_body,
        out_shape=jax.ShapeDtypeStruct((N, ND), jnp.float32),
    )(ms, cnt2, x, wa1, wa2, ba)


def _zscale_body(z_ref, cr_ref, o_ref):
    c2 = cr_ref[...]
    cr = c2[0, :, 0:1] + c2[1, :, 0:1]
    o_ref[...] = z_ref[...] * (1.0 / jnp.maximum(cr, 1.0))[None]


def _tc_zscale(z, cr2):
    B = 2000
    nb = N // B
    return pl.pallas_call(
        _zscale_body,
        grid=(R, nb),
        in_specs=[
            pl.BlockSpec((1, B, ND), lambda r, i: (r, i, 0)),
            pl.BlockSpec((NC, B, ED), lambda r, i: (0, r * nb + i, 0)),
        ],
        out_specs=pl.BlockSpec((1, B, ND), lambda r, i: (r, i, 0)),
        out_shape=jax.ShapeDtypeStruct((R, N, ND), jnp.float32),
    )(z, cr2)


def _relup_body(z_ref, x_ref, wroot_ref, brg_ref, wrel_ref,
                weu_ref, wev_ref, x2_ref, u_ref, v_ref):
    z = z_ref[...]
    acc = jnp.dot(x_ref[...], wroot_ref[...],
                  preferred_element_type=jnp.float32) + brg_ref[...]
    for r in range(R):
        acc = acc + jnp.dot(z[r], wrel_ref[r],
                            preferred_element_type=jnp.float32)
    x2 = jnp.maximum(acc, 0.0)
    x2_ref[...] = x2
    u_ref[...] = jnp.dot(x2, weu_ref[...], preferred_element_type=jnp.float32)
    v_ref[...] = jnp.dot(x2, wev_ref[...], preferred_element_type=jnp.float32)


def _tc_relup(zs, x, wroot, brg, wrel, weu, wev):
    return pl.pallas_call(
        _relup_body,
        out_shape=(jax.ShapeDtypeStruct((N, ND), jnp.float32),
                   jax.ShapeDtypeStruct((N, ED), jnp.float32),
                   jax.ShapeDtypeStruct((N, ED), jnp.float32)),
    )(zs, x, wroot, brg, wrel, weu, wev)


def _final_body(x_ref, w_ref, b_ref, o_ref):
    o_ref[...] = jnp.maximum(
        jnp.dot(x_ref[...], w_ref[...], preferred_element_type=jnp.float32)
        + b_ref[...], 0.0)


def _tc_final(x, w, b):
    return pl.pallas_call(
        _final_body,
        out_shape=jax.ShapeDtypeStruct((N, w.shape[1]), jnp.float32),
    )(x, w, b)


def _pad16(w):
    k, n = w.shape
    if k == ED:
        return w
    return jnp.concatenate([w, jnp.zeros((ED - k, n), jnp.float32)], axis=0)


@jax.jit
def kernel(x, edge_index, edge_attr, rel_edge_index, rel_edge_type, params):
    src2d = edge_index[0].reshape(ROWS, LROW)
    dst2d = edge_index[1].reshape(ROWS, LROW)
    rsrc2d = rel_edge_index[0].reshape(ROWS, LROW)
    rdst2d = rel_edge_index[1].reshape(ROWS, LROW)
    rtyp2d = rel_edge_type.reshape(ROWS, LROW)

    ea16 = jnp.concatenate(
        [edge_attr[:, None], jnp.zeros((E, ED - 1), jnp.float32)], axis=1)

    cnt_flat, crel_flat = _sc_counts(dst2d, rdst2d, rtyp2d)
    cnt2 = cnt_flat.reshape(NC, N, ED)
    crel2 = crel_flat.reshape(NC, 4 * N, ED)

    for l in range(3):
        Wm, bm = params['Wm%d' % l], params['bm%d' % l]
        Wa, ba = params['Wa%d' % l], params['ba%d' % l]
        We, be = params['We%d' % l], params['be%d' % l]
        Wrel = params['Wrel%d' % l]
        Wroot, brg = params['Wroot%d' % l], params['brg%d' % l]
        dx = x.shape[1]

        em = _tc_edge_mm(ea16, _pad16(Wm[dx:]), bm[None, :])
        xm = _tc_mm(x, Wm[:dx])
        ms = _sc_msum(xm, em, src2d, dst2d).reshape(NC, N, ND)
        x1 = _tc_aggup(ms, cnt2, x, Wa[:ND], Wa[ND:], ba[None, :])
        zflat = _sc_rel(x1, rsrc2d, rdst2d, rtyp2d)
        zs = _tc_zscale(zflat.reshape(R, N, ND), crel2)
        x2, u, v = _tc_relup(zs, x1, Wroot, brg[None, :], Wrel,
                             We[:ND], We[ND:2 * ND])
        te = _tc_edge_mm(ea16, _pad16(We[2 * ND:]), be[None, :])
        ea16 = _sc_ea(u, v, te, src2d, dst2d)
        x = x2

    return _tc_final(x, params['Wout'], params['bout'][None, :])
